# baseline (speedup 1.0000x reference)
import jax
import jax.numpy as jnp
from jax.experimental import pallas as pl
from jax.experimental.pallas import tpu as pltpu

_B, _N, _OBS = 64, 512, 128
_H, _OUT, _K = 256, 64, 16


_BPB = 16


def _gnn_body(obs_ref, posT_ref, w1_ref, b1_ref, w2_ref, b2_ref, wo_ref,
              bo_ref, out_ref):
    cidx = jax.lax.broadcasted_iota(jnp.int32, (_N, _N), 0)
    ridx = jax.lax.broadcasted_iota(jnp.int32, (_N, _N), 1)
    _SENT = jnp.float32(3e38)
    for bb in range(_BPB):
        x = obs_ref[bb]
        pxc = x[:, 0:1]
        pyc = x[:, 1:2]
        pxr = posT_ref[bb, 0:1, :]
        pyr = posT_ref[bb, 1:2, :]
        dx = pxr - pxc
        dy = pyr - pyc
        d = jnp.where(cidx == ridx, _SENT, dx * dx + dy * dy)
        m = jnp.min(d, axis=0, keepdims=True)
        for _ in range(_K):
            d = jnp.where(d == m, _SENT, d)
            m = jnp.min(d, axis=0, keepdims=True)

        s = (d >= jnp.float32(2e38)).astype(jnp.float32)
        deg = jnp.sum(s, axis=1, keepdims=True)
        dinv = jax.lax.rsqrt(deg)

        h1 = jnp.dot(x, w1_ref[...], preferred_element_type=jnp.float32)
        g1 = dinv * jnp.dot(s, dinv * h1,
                            preferred_element_type=jnp.float32) + b1_ref[...]
        x1 = jnp.tanh(g1)
        h2 = jnp.dot(x1, w2_ref[...], preferred_element_type=jnp.float32)
        g2 = dinv * jnp.dot(s, dinv * h2,
                            preferred_element_type=jnp.float32) + b2_ref[...]
        x2 = jnp.tanh(g2)
        out_ref[bb] = jnp.dot(x2, wo_ref[...],
                              preferred_element_type=jnp.float32) + bo_ref[...]


@jax.jit
def kernel(agent_observations, W1, b1, W2, b2, W_out, b_out):
    obs = agent_observations.astype(jnp.float32)
    batch, n, obs_dim = obs.shape
    hidden = W1.shape[1]
    out_dim = W_out.shape[1]

    posT = jnp.zeros((batch, 8, n), jnp.float32)
    posT = posT.at[:, 0, :].set(obs[:, :, 0]).at[:, 1, :].set(obs[:, :, 1])

    const = lambda b: (0, 0)
    return pl.pallas_call(
        _gnn_body,
        grid=(batch // _BPB,),
        in_specs=[
            pl.BlockSpec((_BPB, n, obs_dim), lambda b: (b, 0, 0)),
            pl.BlockSpec((_BPB, 8, n), lambda b: (b, 0, 0)),
            pl.BlockSpec((obs_dim, hidden), const),
            pl.BlockSpec((1, hidden), const),
            pl.BlockSpec((hidden, hidden), const),
            pl.BlockSpec((1, hidden), const),
            pl.BlockSpec((hidden, out_dim), const),
            pl.BlockSpec((1, out_dim), const),
        ],
        out_specs=pl.BlockSpec((_BPB, n, out_dim), lambda b: (b, 0, 0)),
        out_shape=jax.ShapeDtypeStruct((batch, n, out_dim), jnp.float32),
        compiler_params=pltpu.CompilerParams(
            dimension_semantics=("arbitrary",),
        ),
    )(obs, posT, W1, b1.reshape(1, hidden), W2, b2.reshape(1, hidden),
      W_out, b_out.reshape(1, out_dim))


# speedup vs baseline: 1.2146x; 1.2146x over previous
import jax
import jax.numpy as jnp
from jax.experimental import pallas as pl
from jax.experimental.pallas import tpu as pltpu

_B, _N, _OBS = 64, 512, 128
_H, _OUT, _K = 256, 64, 16


_BPB = 8


def _gnn_body(obs_ref, posT_ref, w1_ref, b1_ref, w2_ref, b2_ref, wo_ref,
              bo_ref, out_ref):
    cidx = jax.lax.broadcasted_iota(jnp.int32, (_N, _N), 0)
    ridx = jax.lax.broadcasted_iota(jnp.int32, (_N, _N), 1)
    _SENT = jnp.float32(3e38)
    for bb in range(_BPB):
        x = obs_ref[bb]
        pxc = x[:, 0:1]
        pyc = x[:, 1:2]
        pxr = posT_ref[bb, 0:1, :]
        pyr = posT_ref[bb, 1:2, :]
        dx = pxr - pxc
        dy = pyr - pyc
        d = jnp.where(cidx == ridx, _SENT, dx * dx + dy * dy)
        m = jnp.min(d, axis=0, keepdims=True)
        for _ in range(_K):
            d = jnp.where(d == m, _SENT, d)
            m = jnp.min(d, axis=0, keepdims=True)

        s = (d >= jnp.float32(2e38)).astype(jnp.float32)
        deg = jnp.sum(s, axis=1, keepdims=True)
        dinv = jax.lax.rsqrt(deg)

        h1 = jnp.dot(x, w1_ref[...], preferred_element_type=jnp.float32)
        g1 = dinv * jnp.dot(s, dinv * h1,
                            preferred_element_type=jnp.float32) + b1_ref[...]
        x1 = jnp.tanh(g1)
        h2 = jnp.dot(x1, w2_ref[...], preferred_element_type=jnp.float32)
        g2 = dinv * jnp.dot(s, dinv * h2,
                            preferred_element_type=jnp.float32) + b2_ref[...]
        x2 = jnp.tanh(g2)
        out_ref[bb] = jnp.dot(x2, wo_ref[...],
                              preferred_element_type=jnp.float32) + bo_ref[...]


@jax.jit
def kernel(agent_observations, W1, b1, W2, b2, W_out, b_out):
    obs = agent_observations.astype(jnp.float32)
    batch, n, obs_dim = obs.shape
    hidden = W1.shape[1]
    out_dim = W_out.shape[1]

    posT = jnp.zeros((batch, 8, n), jnp.float32)
    posT = posT.at[:, 0, :].set(obs[:, :, 0]).at[:, 1, :].set(obs[:, :, 1])

    const = lambda b: (0, 0)
    return pl.pallas_call(
        _gnn_body,
        grid=(batch // _BPB,),
        in_specs=[
            pl.BlockSpec((_BPB, n, obs_dim), lambda b: (b, 0, 0)),
            pl.BlockSpec((_BPB, 8, n), lambda b: (b, 0, 0)),
            pl.BlockSpec((obs_dim, hidden), const),
            pl.BlockSpec((1, hidden), const),
            pl.BlockSpec((hidden, hidden), const),
            pl.BlockSpec((1, hidden), const),
            pl.BlockSpec((hidden, out_dim), const),
            pl.BlockSpec((1, out_dim), const),
        ],
        out_specs=pl.BlockSpec((_BPB, n, out_dim), lambda b: (b, 0, 0)),
        out_shape=jax.ShapeDtypeStruct((batch, n, out_dim), jnp.float32),
        compiler_params=pltpu.CompilerParams(
            dimension_semantics=("arbitrary",),
        ),
    )(obs, posT, W1, b1.reshape(1, hidden), W2, b2.reshape(1, hidden),
      W_out, b_out.reshape(1, out_dim))
